# baseline (speedup 1.0000x reference)
import functools

import jax
import jax.numpy as jnp
from jax import lax
from jax.experimental import pallas as pl
from jax.experimental.pallas import tpu as pltpu
from jax.experimental.pallas import tpu_sc as plsc

_IDXW = 128
_CHUNK = 512


@functools.lru_cache(maxsize=None)
def _build(b, t, h, d):
    info = plsc.get_sparse_core_info()
    nc, ns, lanes = info.num_cores, info.num_subcores, info.num_lanes
    nw = nc * ns
    n = b * t * h
    assert _CHUNK % lanes == 0 and t % _CHUNK == 0
    pairs = b * h
    assert pairs % nw == 0
    ppw = pairs // nw
    cpp = t // _CHUNK

    mesh = plsc.VectorSubcoreMesh(core_axis_name="c", subcore_axis_name="s")

    @functools.partial(
        pl.kernel,
        mesh=mesh,
        compiler_params=pltpu.CompilerParams(use_tc_tiling_on_sc=False,
                                             needs_layout_passes=False),
        out_type=jax.ShapeDtypeStruct((b, t, h, d), jnp.float32),
        scratch_types=[
            pltpu.VMEM((2, _CHUNK), jnp.int32),
            pltpu.VMEM((2, _CHUNK, d), jnp.float32),
            pltpu.VMEM((lanes,), jnp.int32),
            pltpu.SemaphoreType.DMA,
            pltpu.SemaphoreType.DMA,
        ],
    )
    def gather(ids_hbm, offs_hbm, table_hbm, out_hbm, idx_v, rows_v, offs_v,
               gsem, wsem):
        wid = lax.axis_index("s") * nc + lax.axis_index("c")
        pltpu.sync_copy(offs_hbm, offs_v)

        def pair_coords(pp):
            pair = wid * ppw + pp
            return pair // h, pair % h

        def load_shift(pp, c, slot):
            bi, hi = pair_coords(pp)
            pltpu.sync_copy(ids_hbm.at[bi, hi, pl.ds(c * _CHUNK, _CHUNK)],
                            idx_v.at[slot])
            off = plsc.load_gather(
                offs_v, [lax.broadcast(hi, (lanes,))])

            @pl.loop(0, _CHUNK // lanes)
            def _(i):
                sl = pl.ds(i * lanes, lanes)
                idx_v[slot, sl] = idx_v[slot, sl] + off

        def fire_gather(slot):
            return pltpu.async_copy(table_hbm.at[idx_v.at[slot]],
                                    rows_v.at[slot], gsem)

        def fire_write(pp, c, slot):
            bi, hi = pair_coords(pp)
            return pltpu.async_copy(
                rows_v.at[slot],
                out_hbm.at[bi, pl.ds(c * _CHUNK, _CHUNK), hi, :],
                wsem)

        n_chunks = ppw * cpp
        coords = [(pp, c) for pp in range(ppw) for c in range(cpp)]
        load_shift(*coords[0], 0)
        g = fire_gather(0)
        pending_w = []
        for k, (pp, c) in enumerate(coords):
            slot = k & 1
            if k + 1 < n_chunks:
                load_shift(*coords[k + 1], 1 - slot)
                if pending_w:
                    pending_w.pop(0).wait()
                g_next = fire_gather(1 - slot)
            g.wait()
            pending_w.append(fire_write(pp, c, slot))
            if k + 1 < n_chunks:
                g = g_next
        for w in pending_w:
            w.wait()

    return gather


_TB = 256


@functools.lru_cache(maxsize=None)
def _build_transpose(dd, v):
    info = plsc.get_sparse_core_info()
    nc, ns, lanes = info.num_cores, info.num_subcores, info.num_lanes
    nw = nc * ns
    assert dd == 4 * lanes
    full = v // _TB
    tail = v - full * _TB
    pairs = _TB // 2

    mesh = plsc.VectorSubcoreMesh(core_axis_name="c", subcore_axis_name="s")

    @functools.partial(
        pl.kernel,
        mesh=mesh,
        compiler_params=pltpu.CompilerParams(use_tc_tiling_on_sc=True,
                                             needs_layout_passes=False),
        out_type=jax.ShapeDtypeStruct((v * dd // 128, 128), jnp.float32),
        scratch_types=[
            pltpu.VMEM((2, dd, _TB), jnp.float32),
            pltpu.VMEM((2, pairs, 2 * dd), jnp.float32),
            pltpu.SemaphoreType.DMA,
            pltpu.SemaphoreType.DMA,
        ],
    )
    def transp(tt_hbm, tail_hbm, out_hbm, in_v, out_v, rsem, wsem):
        wid = lax.axis_index("s") * nc + lax.axis_index("c")
        nblk = (full - wid + nw - 1) // nw

        def blk_of(i):
            return wid + i * nw

        def fire_read(i, slot):
            pltpu.async_copy(
                tt_hbm.at[:, pl.ds(blk_of(i) * _TB, _TB)], in_v.at[slot], rsem)

        def fire_write(i, slot):
            pltpu.async_copy(
                out_v.at[slot],
                out_hbm.at[pl.ds(blk_of(i) * pairs, pairs), :], wsem)

        def wait_read(slot):
            pltpu.make_async_copy(
                tt_hbm.at[:, pl.ds(0, _TB)], in_v.at[slot], rsem).wait()

        def wait_write(slot):
            pltpu.make_async_copy(
                out_v.at[slot],
                out_hbm.at[pl.ds(0, pairs), :], wsem).wait()

        rowg = [lax.iota(jnp.int32, lanes) + lanes * g
                for g in range(dd // lanes)]

        def transpose_block(slot):
            @pl.loop(0, pairs, unroll=8)
            def _(pp):
                for half in range(2):
                    col = lax.broadcast(2 * pp + half, (lanes,))
                    for g in range(dd // lanes):
                        vals = plsc.load_gather(in_v.at[slot],
                                                [rowg[g], col])
                        out_v[slot, pp,
                              pl.ds(half * dd + g * lanes, lanes)] = vals

        @pl.when(nblk > 0)
        def _():
            fire_read(0, 0)

            def body(i, carry):
                slot = lax.rem(i, 2)

                @pl.when(i + 1 < nblk)
                def _():
                    fire_read(i + 1, 1 - slot)
                wait_read(slot)

                @pl.when(i >= 2)
                def _():
                    wait_write(slot)
                transpose_block(slot)
                fire_write(i, slot)
                return carry

            lax.fori_loop(0, nblk, body, 0)

            @pl.when(nblk >= 2)
            def _():
                wait_write(lax.rem(nblk, 2))
            wait_write(lax.rem(nblk + 1, 2))

        @pl.when(wid == nw - 1)
        def _():
            pltpu.sync_copy(tail_hbm,
                            out_hbm.at[pl.ds(full * pairs, tail // 2), :])

    return transp


@functools.lru_cache(maxsize=None)
def _build_tiler(n_rows, n_cols, rows_per_step):
    info = plsc.get_sparse_core_info()
    nw = info.num_cores * info.num_subcores
    nc = info.num_cores
    assert n_rows % (nw * rows_per_step) == 0 and n_cols % 128 == 0
    steps = n_rows // (nw * rows_per_step)
    per_w = n_rows // nw
    cpr = n_cols // 128
    in_rps = rows_per_step * cpr

    mesh = plsc.VectorSubcoreMesh(core_axis_name="c", subcore_axis_name="s")

    @functools.partial(
        pl.kernel,
        mesh=mesh,
        compiler_params=pltpu.CompilerParams(use_tc_tiling_on_sc=True),
        out_type=jax.ShapeDtypeStruct((n_rows, n_cols), jnp.float32),
        scratch_types=[
            pltpu.VMEM((2, in_rps, 128), jnp.float32),
            pltpu.SemaphoreType.DMA,
            pltpu.SemaphoreType.DMA,
        ],
    )
    def tile_out(in_hbm, out_hbm, buf, rsem, wsem):
        wid = lax.axis_index("s") * nc + lax.axis_index("c")
        row0 = wid * per_w

        def read(c, slot):
            return pltpu.async_copy(
                in_hbm.at[pl.ds((row0 + c * rows_per_step) * cpr, in_rps), :],
                buf.at[slot], rsem)

        r = read(0, 0)
        pending_w = []
        for c in range(steps):
            slot = c & 1
            if c + 1 < steps:
                if pending_w:
                    pending_w.pop(0).wait()
                r_next = read(c + 1, 1 - slot)
            r.wait()
            pending_w.append(pltpu.async_copy(
                buf.at[slot].reshape(rows_per_step, n_cols),
                out_hbm.at[pl.ds((row0 + c * rows_per_step), rows_per_step), :],
                wsem))
            if c + 1 < steps:
                r = r_next
        for w in pending_w:
            w.wait()

    return tile_out


def kernel(input_ids, offsets, table):
    b, t, h = input_ids.shape
    d = table.shape[1]
    n = b * t * h
    ids_bht = input_ids.transpose(0, 2, 1)
    v, d_ = table.shape
    full = v // _TB
    tableT = table.T
    tail2 = table[full * _TB:, :].reshape((v - full * _TB) * d_ // 128, 128)
    lin = _build_transpose(d_, v)(tableT, tail2)
    out = _build(b, t, h, d)(ids_bht, offsets, lin.reshape(v, d_))
    out_tiled = _build_tiler(b * t, h * d, 16)(out.reshape(n * d // 128, 128))

    return out_tiled.reshape(b, t, h * d)


# speedup vs baseline: 1.1981x; 1.1981x over previous
import functools

import jax
import jax.numpy as jnp
from jax import lax
from jax.experimental import pallas as pl
from jax.experimental.pallas import tpu as pltpu
from jax.experimental.pallas import tpu_sc as plsc

_IDXW = 128
_CHUNK = 512


@functools.lru_cache(maxsize=None)
def _build(b, t, h, d):
    info = plsc.get_sparse_core_info()
    nc, ns, lanes = info.num_cores, info.num_subcores, info.num_lanes
    nw = nc * ns
    n = b * t * h
    assert _CHUNK % lanes == 0 and t % _CHUNK == 0
    pairs = b * h
    assert pairs % nw == 0
    ppw = pairs // nw
    cpp = t // _CHUNK

    mesh = plsc.VectorSubcoreMesh(core_axis_name="c", subcore_axis_name="s")

    @functools.partial(
        pl.kernel,
        mesh=mesh,
        compiler_params=pltpu.CompilerParams(use_tc_tiling_on_sc=False,
                                             needs_layout_passes=False),
        out_type=jax.ShapeDtypeStruct((b, t, h, d), jnp.float32),
        scratch_types=[
            pltpu.VMEM((2, _CHUNK), jnp.int32),
            pltpu.VMEM((2, _CHUNK, d), jnp.float32),
            pltpu.VMEM((lanes,), jnp.int32),
            pltpu.SemaphoreType.DMA,
            pltpu.SemaphoreType.DMA,
        ],
    )
    def gather(ids_hbm, offs_hbm, table_hbm, out_hbm, idx_v, rows_v, offs_v,
               gsem, wsem):
        wid = lax.axis_index("s") * nc + lax.axis_index("c")
        pltpu.sync_copy(offs_hbm, offs_v)

        def pair_coords(pp):
            pair = wid * ppw + pp
            return pair // h, pair % h

        def load_shift(pp, c, slot):
            bi, hi = pair_coords(pp)
            pltpu.sync_copy(ids_hbm.at[bi, hi, pl.ds(c * _CHUNK, _CHUNK)],
                            idx_v.at[slot])
            off = plsc.load_gather(
                offs_v, [lax.broadcast(hi, (lanes,))])

            @pl.loop(0, _CHUNK // lanes)
            def _(i):
                sl = pl.ds(i * lanes, lanes)
                idx_v[slot, sl] = idx_v[slot, sl] + off

        def fire_gather(slot):
            return pltpu.async_copy(table_hbm.at[idx_v.at[slot]],
                                    rows_v.at[slot], gsem)

        def fire_write(pp, c, slot):
            bi, hi = pair_coords(pp)
            return pltpu.async_copy(
                rows_v.at[slot],
                out_hbm.at[bi, pl.ds(c * _CHUNK, _CHUNK), hi, :],
                wsem)

        n_chunks = ppw * cpp
        coords = [(pp, c) for pp in range(ppw) for c in range(cpp)]
        load_shift(*coords[0], 0)
        g = fire_gather(0)
        pending_w = []
        for k, (pp, c) in enumerate(coords):
            slot = k & 1
            if k + 1 < n_chunks:
                load_shift(*coords[k + 1], 1 - slot)
                if pending_w:
                    pending_w.pop(0).wait()
                g_next = fire_gather(1 - slot)
            g.wait()
            pending_w.append(fire_write(pp, c, slot))
            if k + 1 < n_chunks:
                g = g_next
        for w in pending_w:
            w.wait()

    return gather


_TB = 128


@functools.lru_cache(maxsize=None)
def _build_transpose(dd, v):
    info = plsc.get_sparse_core_info()
    nc, ns, lanes = info.num_cores, info.num_subcores, info.num_lanes
    nw = nc * ns
    full = v // _TB
    tail = v - full * _TB
    blk_elems = dd * _TB

    mesh = plsc.VectorSubcoreMesh(core_axis_name="c", subcore_axis_name="s")

    @functools.partial(
        pl.kernel,
        mesh=mesh,
        compiler_params=pltpu.CompilerParams(use_tc_tiling_on_sc=True,
                                             needs_layout_passes=False),
        out_type=jax.ShapeDtypeStruct((v * dd,), jnp.float32),
        scratch_types=[
            pltpu.VMEM((dd, _TB), jnp.float32),
            pltpu.VMEM((dd, _TB), jnp.float32),
            pltpu.VMEM((blk_elems,), jnp.float32),
            pltpu.VMEM((blk_elems,), jnp.float32),
            pltpu.SemaphoreType.DMA,
            pltpu.SemaphoreType.DMA,
        ],
    )
    def transp(tt_hbm, tail_hbm, out_hbm, in0, in1, out0, out1, rsem, wsem):
        wid = lax.axis_index("s") * nc + lax.axis_index("c")
        nblk = (full - wid + nw - 1) // nw

        def blk_of(i):
            return wid + i * nw

        def fire_read(i, in_v):
            pltpu.async_copy(
                tt_hbm.at[:, pl.ds(blk_of(i) * _TB, _TB)], in_v, rsem)

        def fire_write(i, out_v):
            pltpu.async_copy(
                out_v, out_hbm.at[pl.ds(blk_of(i) * blk_elems, blk_elems)],
                wsem)

        def wait_read(in_v):
            pltpu.make_async_copy(
                tt_hbm.at[:, pl.ds(0, _TB)], in_v, rsem).wait()

        def wait_write(out_v):
            pltpu.make_async_copy(
                out_v, out_hbm.at[pl.ds(0, blk_elems)], wsem).wait()

        lane_step = lax.iota(jnp.int32, lanes) * dd

        def transpose_block(in_v, out_v):
            for d in range(dd):
                for c0 in range(0, _TB, lanes):
                    vals = in_v[d, pl.ds(c0, lanes)]
                    plsc.store_scatter(out_v, [lane_step + (c0 * dd + d)],
                                       vals)

        fire_read(0, in0)

        @pl.when(nblk > 1)
        def _():
            fire_read(1, in1)

        def body(j, carry):
            wait_read(in0)

            @pl.when(j >= 1)
            def _():
                wait_write(out0)
            transpose_block(in0, out0)
            fire_write(2 * j, out0)

            @pl.when(2 * j + 2 < nblk)
            def _():
                fire_read(2 * j + 2, in0)

            @pl.when(2 * j + 1 < nblk)
            def _():
                wait_read(in1)

                @pl.when(j >= 1)
                def _():
                    wait_write(out1)
                transpose_block(in1, out1)
                fire_write(2 * j + 1, out1)

                @pl.when(2 * j + 3 < nblk)
                def _():
                    fire_read(2 * j + 3, in1)
            return carry

        lax.fori_loop(0, (nblk + 1) // 2, body, 0)
        wait_write(out0)

        @pl.when(nblk > 1)
        def _():
            wait_write(out1)

        @pl.when(wid == nw - 1)
        def _():
            pltpu.sync_copy(tail_hbm,
                            out_hbm.at[pl.ds(full * blk_elems, tail * dd)])

    return transp


@functools.lru_cache(maxsize=None)
def _build_tiler(n_rows, n_cols, rows_per_step):
    info = plsc.get_sparse_core_info()
    nw = info.num_cores * info.num_subcores
    nc = info.num_cores
    assert n_rows % (nw * rows_per_step) == 0 and n_cols % 128 == 0
    steps = n_rows // (nw * rows_per_step)
    per_w = n_rows // nw
    cpr = n_cols // 128
    in_rps = rows_per_step * cpr

    mesh = plsc.VectorSubcoreMesh(core_axis_name="c", subcore_axis_name="s")

    @functools.partial(
        pl.kernel,
        mesh=mesh,
        compiler_params=pltpu.CompilerParams(use_tc_tiling_on_sc=True),
        out_type=jax.ShapeDtypeStruct((n_rows, n_cols), jnp.float32),
        scratch_types=[
            pltpu.VMEM((2, in_rps, 128), jnp.float32),
            pltpu.SemaphoreType.DMA,
            pltpu.SemaphoreType.DMA,
        ],
    )
    def tile_out(in_hbm, out_hbm, buf, rsem, wsem):
        wid = lax.axis_index("s") * nc + lax.axis_index("c")
        row0 = wid * per_w

        def read(c, slot):
            return pltpu.async_copy(
                in_hbm.at[pl.ds((row0 + c * rows_per_step) * cpr, in_rps), :],
                buf.at[slot], rsem)

        r = read(0, 0)
        pending_w = []
        for c in range(steps):
            slot = c & 1
            if c + 1 < steps:
                if pending_w:
                    pending_w.pop(0).wait()
                r_next = read(c + 1, 1 - slot)
            r.wait()
            pending_w.append(pltpu.async_copy(
                buf.at[slot].reshape(rows_per_step, n_cols),
                out_hbm.at[pl.ds((row0 + c * rows_per_step), rows_per_step), :],
                wsem))
            if c + 1 < steps:
                r = r_next
        for w in pending_w:
            w.wait()

    return tile_out


def kernel(input_ids, offsets, table):
    b, t, h = input_ids.shape
    d = table.shape[1]
    n = b * t * h
    ids_bht = input_ids.transpose(0, 2, 1)
    v, d_ = table.shape
    full = v // _TB
    tableT = table.T
    tail2 = table[full * _TB:, :].reshape((v - full * _TB) * d_)
    lin = _build_transpose(d_, v)(tableT, tail2)
    out = _build(b, t, h, d)(ids_bht, offsets, lin.reshape(v, d_))
    out_tiled = _build_tiler(b * t, h * d, 16)(out.reshape(n * d // 128, 128))

    return out_tiled.reshape(b, t, h * d)


# speedup vs baseline: 1.2264x; 1.0237x over previous
import functools

import jax
import jax.numpy as jnp
from jax import lax
from jax.experimental import pallas as pl
from jax.experimental.pallas import tpu as pltpu
from jax.experimental.pallas import tpu_sc as plsc

_IDXW = 128
_CHUNK = 512


@functools.lru_cache(maxsize=None)
def _build(b, t, h, d):
    info = plsc.get_sparse_core_info()
    nc, ns, lanes = info.num_cores, info.num_subcores, info.num_lanes
    nw = nc * ns
    n = b * t * h
    assert _CHUNK % lanes == 0 and t % _CHUNK == 0
    pairs = b * h
    assert pairs % nw == 0
    ppw = pairs // nw
    cpp = t // _CHUNK

    mesh = plsc.VectorSubcoreMesh(core_axis_name="c", subcore_axis_name="s")

    @functools.partial(
        pl.kernel,
        mesh=mesh,
        compiler_params=pltpu.CompilerParams(use_tc_tiling_on_sc=False,
                                             needs_layout_passes=False),
        out_type=jax.ShapeDtypeStruct((b, t, h, d), jnp.float32),
        scratch_types=[
            pltpu.VMEM((2, _CHUNK), jnp.int32),
            pltpu.VMEM((2, _CHUNK, d), jnp.float32),
            pltpu.VMEM((lanes,), jnp.int32),
            pltpu.SemaphoreType.DMA,
            pltpu.SemaphoreType.DMA,
        ],
    )
    def gather(ids_hbm, offs_hbm, table_hbm, out_hbm, idx_v, rows_v, offs_v,
               gsem, wsem):
        wid = lax.axis_index("s") * nc + lax.axis_index("c")
        pltpu.sync_copy(offs_hbm, offs_v)

        def pair_coords(pp):
            pair = wid * ppw + pp
            return pair // h, pair % h

        def load_shift(pp, c, slot):
            bi, hi = pair_coords(pp)
            pltpu.sync_copy(ids_hbm.at[bi, hi, pl.ds(c * _CHUNK, _CHUNK)],
                            idx_v.at[slot])
            off = plsc.load_gather(
                offs_v, [lax.broadcast(hi, (lanes,))])

            @pl.loop(0, _CHUNK // lanes)
            def _(i):
                sl = pl.ds(i * lanes, lanes)
                idx_v[slot, sl] = idx_v[slot, sl] + off

        def fire_gather(slot):
            return pltpu.async_copy(table_hbm.at[idx_v.at[slot]],
                                    rows_v.at[slot], gsem)

        def fire_write(pp, c, slot):
            bi, hi = pair_coords(pp)
            return pltpu.async_copy(
                rows_v.at[slot],
                out_hbm.at[bi, pl.ds(c * _CHUNK, _CHUNK), hi, :],
                wsem)

        n_chunks = ppw * cpp
        coords = [(pp, c) for pp in range(ppw) for c in range(cpp)]
        load_shift(*coords[0], 0)
        g = fire_gather(0)
        pending_w = []
        for k, (pp, c) in enumerate(coords):
            slot = k & 1
            if k + 1 < n_chunks:
                load_shift(*coords[k + 1], 1 - slot)
                if pending_w:
                    pending_w.pop(0).wait()
                g_next = fire_gather(1 - slot)
            g.wait()
            pending_w.append(fire_write(pp, c, slot))
            if k + 1 < n_chunks:
                g = g_next
        for w in pending_w:
            w.wait()

    return gather


_TB = 128


@functools.lru_cache(maxsize=None)
def _build_transpose(dd, v):
    info = plsc.get_sparse_core_info()
    nc, ns, lanes = info.num_cores, info.num_subcores, info.num_lanes
    nw = nc * ns
    full = v // _TB
    tail = v - full * _TB
    blk_elems = dd * _TB

    mesh = plsc.VectorSubcoreMesh(core_axis_name="c", subcore_axis_name="s")

    @functools.partial(
        pl.kernel,
        mesh=mesh,
        compiler_params=pltpu.CompilerParams(use_tc_tiling_on_sc=True,
                                             needs_layout_passes=False),
        out_type=jax.ShapeDtypeStruct((v * dd,), jnp.float32),
        scratch_types=[
            pltpu.VMEM((dd, _TB), jnp.float32),
            pltpu.VMEM((dd, _TB), jnp.float32),
            pltpu.VMEM((blk_elems,), jnp.float32),
            pltpu.VMEM((blk_elems,), jnp.float32),
            pltpu.SemaphoreType.DMA,
            pltpu.SemaphoreType.DMA,
        ],
    )
    def transp(tt_hbm, tail_hbm, out_hbm, in0, in1, out0, out1, rsem, wsem):
        wid = lax.axis_index("s") * nc + lax.axis_index("c")
        nblk = (full - wid + nw - 1) // nw

        def blk_of(i):
            return wid + i * nw

        def fire_read(i, in_v):
            pltpu.async_copy(
                tt_hbm.at[:, pl.ds(blk_of(i) * _TB, _TB)], in_v, rsem)

        def fire_write(i, out_v):
            pltpu.async_copy(
                out_v, out_hbm.at[pl.ds(blk_of(i) * blk_elems, blk_elems)],
                wsem)

        def wait_read(in_v):
            pltpu.make_async_copy(
                tt_hbm.at[:, pl.ds(0, _TB)], in_v, rsem).wait()

        def wait_write(out_v):
            pltpu.make_async_copy(
                out_v, out_hbm.at[pl.ds(0, blk_elems)], wsem).wait()

        lane_step = lax.iota(jnp.int32, lanes) * dd

        def transpose_block(in_v, out_v):
            groups = _TB // lanes
            for d in range(dd):
                vals = [in_v[d, pl.ds(c0 * lanes, lanes)]
                        for c0 in range(groups)]
                for c0 in range(groups):
                    plsc.store_scatter(
                        out_v, [lane_step + (c0 * lanes * dd + d)], vals[c0])

        fire_read(0, in0)

        @pl.when(nblk > 1)
        def _():
            fire_read(1, in1)

        def body(j, carry):
            wait_read(in0)

            @pl.when(j >= 1)
            def _():
                wait_write(out0)
            transpose_block(in0, out0)
            fire_write(2 * j, out0)

            @pl.when(2 * j + 2 < nblk)
            def _():
                fire_read(2 * j + 2, in0)

            @pl.when(2 * j + 1 < nblk)
            def _():
                wait_read(in1)

                @pl.when(j >= 1)
                def _():
                    wait_write(out1)
                transpose_block(in1, out1)
                fire_write(2 * j + 1, out1)

                @pl.when(2 * j + 3 < nblk)
                def _():
                    fire_read(2 * j + 3, in1)
            return carry

        lax.fori_loop(0, (nblk + 1) // 2, body, 0)
        wait_write(out0)

        @pl.when(nblk > 1)
        def _():
            wait_write(out1)

        @pl.when(wid == nw - 1)
        def _():
            pltpu.sync_copy(tail_hbm,
                            out_hbm.at[pl.ds(full * blk_elems, tail * dd)])

    return transp


@functools.lru_cache(maxsize=None)
def _build_tiler(n_rows, n_cols, rows_per_step):
    info = plsc.get_sparse_core_info()
    nw = info.num_cores * info.num_subcores
    nc = info.num_cores
    assert n_rows % (nw * rows_per_step) == 0 and n_cols % 128 == 0
    steps = n_rows // (nw * rows_per_step)
    per_w = n_rows // nw
    cpr = n_cols // 128
    in_rps = rows_per_step * cpr

    mesh = plsc.VectorSubcoreMesh(core_axis_name="c", subcore_axis_name="s")

    @functools.partial(
        pl.kernel,
        mesh=mesh,
        compiler_params=pltpu.CompilerParams(use_tc_tiling_on_sc=True),
        out_type=jax.ShapeDtypeStruct((n_rows, n_cols), jnp.float32),
        scratch_types=[
            pltpu.VMEM((2, in_rps, 128), jnp.float32),
            pltpu.SemaphoreType.DMA,
            pltpu.SemaphoreType.DMA,
        ],
    )
    def tile_out(in_hbm, out_hbm, buf, rsem, wsem):
        wid = lax.axis_index("s") * nc + lax.axis_index("c")
        row0 = wid * per_w

        def read(c, slot):
            return pltpu.async_copy(
                in_hbm.at[pl.ds((row0 + c * rows_per_step) * cpr, in_rps), :],
                buf.at[slot], rsem)

        r = read(0, 0)
        pending_w = []
        for c in range(steps):
            slot = c & 1
            if c + 1 < steps:
                if pending_w:
                    pending_w.pop(0).wait()
                r_next = read(c + 1, 1 - slot)
            r.wait()
            pending_w.append(pltpu.async_copy(
                buf.at[slot].reshape(rows_per_step, n_cols),
                out_hbm.at[pl.ds((row0 + c * rows_per_step), rows_per_step), :],
                wsem))
            if c + 1 < steps:
                r = r_next
        for w in pending_w:
            w.wait()

    return tile_out


def kernel(input_ids, offsets, table):
    b, t, h = input_ids.shape
    d = table.shape[1]
    n = b * t * h
    ids_bht = input_ids.transpose(0, 2, 1)
    v, d_ = table.shape
    full = v // _TB
    tableT = table.T
    tail2 = table[full * _TB:, :].reshape((v - full * _TB) * d_)
    lin = _build_transpose(d_, v)(tableT, tail2)
    out = _build(b, t, h, d)(ids_bht, offsets, lin.reshape(v, d_))
    out_tiled = _build_tiler(b * t, h * d, 16)(out.reshape(n * d // 128, 128))

    return out_tiled.reshape(b, t, h * d)


# speedup vs baseline: 2.3530x; 1.9186x over previous
import functools

import jax
import jax.numpy as jnp
from jax import lax
from jax.experimental import pallas as pl
from jax.experimental.pallas import tpu as pltpu
from jax.experimental.pallas import tpu_sc as plsc

_IDXW = 128
_CHUNK = 512


@functools.lru_cache(maxsize=None)
def _build(b, t, h, d):
    info = plsc.get_sparse_core_info()
    nc, ns, lanes = info.num_cores, info.num_subcores, info.num_lanes
    nw = nc * ns
    n = b * t * h
    assert _CHUNK % lanes == 0 and t % _CHUNK == 0
    pairs = b * h
    assert pairs % nw == 0
    ppw = pairs // nw
    cpp = t // _CHUNK

    mesh = plsc.VectorSubcoreMesh(core_axis_name="c", subcore_axis_name="s")

    @functools.partial(
        pl.kernel,
        mesh=mesh,
        compiler_params=pltpu.CompilerParams(use_tc_tiling_on_sc=False,
                                             needs_layout_passes=False),
        out_type=jax.ShapeDtypeStruct((b, t, h, d), jnp.float32),
        scratch_types=[
            pltpu.VMEM((2, _CHUNK), jnp.int32),
            pltpu.VMEM((2, _CHUNK, d), jnp.float32),
            pltpu.VMEM((lanes,), jnp.int32),
            pltpu.SemaphoreType.DMA,
            pltpu.SemaphoreType.DMA,
        ],
    )
    def gather(ids_hbm, offs_hbm, table_hbm, out_hbm, idx_v, rows_v, offs_v,
               gsem, wsem):
        wid = lax.axis_index("s") * nc + lax.axis_index("c")
        pltpu.sync_copy(offs_hbm, offs_v)

        def pair_coords(pp):
            pair = wid * ppw + pp
            return pair // h, pair % h

        def load_shift(pp, c, slot):
            bi, hi = pair_coords(pp)
            pltpu.sync_copy(ids_hbm.at[bi, hi, pl.ds(c * _CHUNK, _CHUNK)],
                            idx_v.at[slot])
            off = plsc.load_gather(
                offs_v, [lax.broadcast(hi, (lanes,))])

            @pl.loop(0, _CHUNK // lanes)
            def _(i):
                sl = pl.ds(i * lanes, lanes)
                idx_v[slot, sl] = idx_v[slot, sl] + off

        def fire_gather(slot):
            return pltpu.async_copy(table_hbm.at[idx_v.at[slot]],
                                    rows_v.at[slot], gsem)

        def fire_write(pp, c, slot):
            bi, hi = pair_coords(pp)
            return pltpu.async_copy(
                rows_v.at[slot],
                out_hbm.at[bi, pl.ds(c * _CHUNK, _CHUNK), hi, :],
                wsem)

        n_chunks = ppw * cpp
        coords = [(pp, c) for pp in range(ppw) for c in range(cpp)]
        load_shift(*coords[0], 0)
        g = fire_gather(0)
        pending_w = []
        for k, (pp, c) in enumerate(coords):
            slot = k & 1
            if k + 1 < n_chunks:
                load_shift(*coords[k + 1], 1 - slot)
                if pending_w:
                    pending_w.pop(0).wait()
                g_next = fire_gather(1 - slot)
            g.wait()
            pending_w.append(fire_write(pp, c, slot))
            if k + 1 < n_chunks:
                g = g_next
        for w in pending_w:
            w.wait()

    return gather


@functools.lru_cache(maxsize=None)
def _build_tiler(n_rows, n_cols, rows_per_step):
    info = plsc.get_sparse_core_info()
    nw = info.num_cores * info.num_subcores
    nc = info.num_cores
    assert n_rows % (nw * rows_per_step) == 0 and n_cols % 128 == 0
    steps = n_rows // (nw * rows_per_step)
    per_w = n_rows // nw
    cpr = n_cols // 128
    in_rps = rows_per_step * cpr

    mesh = plsc.VectorSubcoreMesh(core_axis_name="c", subcore_axis_name="s")

    @functools.partial(
        pl.kernel,
        mesh=mesh,
        compiler_params=pltpu.CompilerParams(use_tc_tiling_on_sc=True),
        out_type=jax.ShapeDtypeStruct((n_rows, n_cols), jnp.float32),
        scratch_types=[
            pltpu.VMEM((2, in_rps, 128), jnp.float32),
            pltpu.SemaphoreType.DMA,
            pltpu.SemaphoreType.DMA,
        ],
    )
    def tile_out(in_hbm, out_hbm, buf, rsem, wsem):
        wid = lax.axis_index("s") * nc + lax.axis_index("c")
        row0 = wid * per_w

        def read(c, slot):
            return pltpu.async_copy(
                in_hbm.at[pl.ds((row0 + c * rows_per_step) * cpr, in_rps), :],
                buf.at[slot], rsem)

        r = read(0, 0)
        pending_w = []
        for c in range(steps):
            slot = c & 1
            if c + 1 < steps:
                if pending_w:
                    pending_w.pop(0).wait()
                r_next = read(c + 1, 1 - slot)
            r.wait()
            pending_w.append(pltpu.async_copy(
                buf.at[slot].reshape(rows_per_step, n_cols),
                out_hbm.at[pl.ds((row0 + c * rows_per_step), rows_per_step), :],
                wsem))
            if c + 1 < steps:
                r = r_next
        for w in pending_w:
            w.wait()

    return tile_out


def kernel(input_ids, offsets, table):
    b, t, h = input_ids.shape
    d = table.shape[1]
    n = b * t * h
    ids_bht = input_ids.transpose(0, 2, 1)
    v, d_ = table.shape
    table2 = lax.optimization_barrier(table.reshape(v * d_ // 128, 128))
    out = _build(b, t, h, d)(ids_bht, offsets, table2.reshape(v, d_))
    out_tiled = _build_tiler(b * t, h * d, 16)(out.reshape(n * d // 128, 128))

    return out_tiled.reshape(b, t, h * d)
